# baseline (speedup 1.0000x reference)
import jax
import jax.numpy as jnp
from jax.experimental import pallas as pl
from jax.experimental.pallas import tpu as pltpu

N_EXPERT = 16
DIM_IN = 2048
TILE = 2048


def _gating_kernel(x_ref, w_ref, cw_ref, ei_ref):
    x = x_ref[...]
    w = w_ref[...]
    logits = jax.lax.dot_general(
        x, w, (((1,), (1,)), ((), ())), preferred_element_type=jnp.float32
    )
    lt = logits.T
    t = lt.shape[1]
    iota = jax.lax.broadcasted_iota(jnp.int32, (N_EXPERT, t), 0)

    m1 = jnp.max(lt, axis=0, keepdims=True)
    idx1 = jnp.min(
        jnp.where(lt == m1, iota, N_EXPERT), axis=0, keepdims=True
    )
    masked = jnp.where(iota == idx1, -jnp.inf, lt)
    m2 = jnp.max(masked, axis=0, keepdims=True)
    idx2 = jnp.min(
        jnp.where(masked == m2, iota, N_EXPERT), axis=0, keepdims=True
    )

    z = jnp.sum(jnp.exp(lt - m1), axis=0, keepdims=True)
    p1 = 1.0 / z
    p2 = jnp.exp(m2 - m1) / z
    den = p1 + p2 + 1e-09
    cwt = jnp.concatenate([p1 / den, p2 / den], axis=0)
    eit = jnp.concatenate([idx1, idx2], axis=0)
    cw_ref[...] = cwt.T
    ei_ref[...] = eit.T


def kernel(x, W):
    b, n, d = x.shape
    tokens = b * n
    xf = x.reshape(tokens, d)
    grid = (tokens // TILE,)
    cw, ei = pl.pallas_call(
        _gating_kernel,
        grid=grid,
        in_specs=[
            pl.BlockSpec((TILE, d), lambda i: (i, 0)),
            pl.BlockSpec((N_EXPERT, d), lambda i: (0, 0)),
        ],
        out_specs=[
            pl.BlockSpec((TILE, 2), lambda i: (i, 0)),
            pl.BlockSpec((TILE, 2), lambda i: (i, 0)),
        ],
        out_shape=[
            jax.ShapeDtypeStruct((tokens, 2), jnp.float32),
            jax.ShapeDtypeStruct((tokens, 2), jnp.int32),
        ],
        compiler_params=pltpu.CompilerParams(
            dimension_semantics=("parallel",),
        ),
    )(xf, W)
    return cw.reshape(b, n, 2), ei.reshape(b, n, 2)


# speedup vs baseline: 1.0284x; 1.0284x over previous
import jax
import jax.numpy as jnp
from jax.experimental import pallas as pl
from jax.experimental.pallas import tpu as pltpu

N_EXPERT = 16
DIM_IN = 2048
TILE = 1024


def _gating_kernel(x_ref, w_ref, cw_ref, ei_ref):
    x = x_ref[...]
    w = w_ref[...]
    logits = jax.lax.dot_general(
        x, w, (((1,), (1,)), ((), ())), preferred_element_type=jnp.float32
    )
    lt = logits.T
    t = lt.shape[1]
    iota = jax.lax.broadcasted_iota(jnp.int32, (N_EXPERT, t), 0)

    m1 = jnp.max(lt, axis=0, keepdims=True)
    idx1 = jnp.min(
        jnp.where(lt == m1, iota, N_EXPERT), axis=0, keepdims=True
    )
    masked = jnp.where(iota == idx1, -jnp.inf, lt)
    m2 = jnp.max(masked, axis=0, keepdims=True)
    idx2 = jnp.min(
        jnp.where(masked == m2, iota, N_EXPERT), axis=0, keepdims=True
    )

    z = jnp.sum(jnp.exp(lt - m1), axis=0, keepdims=True)
    p1 = 1.0 / z
    p2 = jnp.exp(m2 - m1) / z
    den = p1 + p2 + 1e-09
    cwt = jnp.concatenate([p1 / den, p2 / den], axis=0)
    eit = jnp.concatenate([idx1, idx2], axis=0)
    cw_ref[...] = cwt.T
    ei_ref[...] = eit.T


def kernel(x, W):
    b, n, d = x.shape
    tokens = b * n
    xf = x.reshape(tokens, d)
    grid = (tokens // TILE,)
    cw, ei = pl.pallas_call(
        _gating_kernel,
        grid=grid,
        in_specs=[
            pl.BlockSpec((TILE, d), lambda i: (i, 0)),
            pl.BlockSpec((N_EXPERT, d), lambda i: (0, 0)),
        ],
        out_specs=[
            pl.BlockSpec((TILE, 2), lambda i: (i, 0)),
            pl.BlockSpec((TILE, 2), lambda i: (i, 0)),
        ],
        out_shape=[
            jax.ShapeDtypeStruct((tokens, 2), jnp.float32),
            jax.ShapeDtypeStruct((tokens, 2), jnp.int32),
        ],
        compiler_params=pltpu.CompilerParams(
            dimension_semantics=("parallel",),
        ),
    )(xf, W)
    return cw.reshape(b, n, 2), ei.reshape(b, n, 2)
